# Initial kernel scaffold; baseline (speedup 1.0000x reference)
import jax
import jax.numpy as jnp
from jax.experimental import pallas as pl


def kernel(x, edge_index, batch, Wrel0, brel0, Wroot0, Wrel1, brel1, Wroot1, Wrel2, brel2, Wroot2, Wlin, blin):
    raise NotImplementedError("write your pallas kernel here")


# speedup vs baseline: 3.0225x; 3.0225x over previous
import jax
import jax.numpy as jnp
from jax import lax
from jax.experimental import pallas as pl
from jax.experimental.pallas import tpu as pltpu
from jax.experimental.pallas import tpu_sc as plsc

N = 10000
E = 320000
D = 128
HD = D // 2
G = 128

NUM_TILES = 32
CHUNK = 128
CH_PER_TILE = 80
EDGES_PER_TILE = CHUNK * CH_PER_TILE
E_PAD = NUM_TILES * EDGES_PER_TILE
ROWS_PER_TILE = 640
N_ACC = 16 * ROWS_PER_TILE


def _sc_segsum_body(h_hbm, src_hbm, dst_hbm, out_hbm,
                    src_v, dst_v, rows_v, agg_sh, sem0, sem1):
    c = lax.axis_index("c")
    s = lax.axis_index("s")
    wid = c * 16 + s
    base = s * ROWS_PER_TILE

    pltpu.sync_copy(src_hbm.at[wid], src_v)
    pltpu.sync_copy(dst_hbm.at[wid], dst_v)

    sems = (sem0, sem1)

    def _gather_start(hf, j, b):
        pltpu.make_async_copy(hf.at[src_v.at[j]], rows_v.at[b], sems[b]).start()

    def _gather_wait(hf, j, b):
        pltpu.make_async_copy(hf.at[src_v.at[j]], rows_v.at[b], sems[b]).wait()

    for f in range(2):
        hf = h_hbm.at[f]

        def _zero(i, _):
            rows_v[0, i // (HD // 16), pl.ds((i % (HD // 16)) * 16, 16)] = (
                jnp.zeros((16,), jnp.float32))
            return 0

        lax.fori_loop(0, CHUNK * (HD // 16), _zero, 0)
        for k in range(ROWS_PER_TILE // CHUNK):
            pltpu.sync_copy(rows_v.at[0],
                            agg_sh.at[pl.ds(base + k * CHUNK, CHUNK)])
        plsc.subcore_barrier()

        _gather_start(hf, 0, 0)
        _gather_start(hf, 1, 1)

        def _body(i, _):
            for b in range(2):
                j = 2 * i + b
                _gather_wait(hf, j, b)
                pltpu.sync_copy(rows_v.at[b], agg_sh.at[dst_v.at[j]], add=True)

                @pl.when(j + 2 < CH_PER_TILE)
                def _():
                    _gather_start(hf, j + 2, b)
            return 0

        lax.fori_loop(0, CH_PER_TILE // 2, _body, 0)

        plsc.subcore_barrier()
        pltpu.sync_copy(agg_sh.at[pl.ds(base, ROWS_PER_TILE)],
                        out_hbm.at[c, f, pl.ds(base, ROWS_PER_TILE)])


def _sc_segsum(h_split, src_r, dst_r):
    kern = pl.kernel(
        _sc_segsum_body,
        out_type=jax.ShapeDtypeStruct((2, 2, N_ACC, HD), jnp.float32),
        mesh=plsc.VectorSubcoreMesh(core_axis_name="c", subcore_axis_name="s"),
        compiler_params=pltpu.CompilerParams(use_tc_tiling_on_sc=False),
        scratch_types=[
            pltpu.VMEM((CH_PER_TILE, CHUNK), jnp.int32),
            pltpu.VMEM((CH_PER_TILE, CHUNK), jnp.int32),
            pltpu.VMEM((2, CHUNK, HD), jnp.float32),
            pltpu.VMEM_SHARED((N_ACC, HD), jnp.float32),
            pltpu.SemaphoreType.DMA,
            pltpu.SemaphoreType.DMA,
        ],
    )
    return kern(h_split, src_r, dst_r)


BLK = 1000


def _bdot(a, b):
    return jnp.dot(a.astype(jnp.bfloat16), b.astype(jnp.bfloat16),
                   preferred_element_type=jnp.float32)


def _assemble(p_ref, h_ref, wrel_t_ref, brel_ref, wroot_t_ref):
    agg = jnp.concatenate([p_ref[0, 0] + p_ref[1, 0],
                           p_ref[0, 1] + p_ref[1, 1]], axis=-1)
    h = jnp.concatenate([h_ref[0], h_ref[1]], axis=-1)
    acc = _bdot(agg, wrel_t_ref[...]) + brel_ref[...] + _bdot(h, wroot_t_ref[...])
    return jnp.tanh(acc)


def _tc_layer_body(p_ref, h_ref, wrel_t_ref, brel_ref, wroot_t_ref, o_ref):
    res = _assemble(p_ref, h_ref, wrel_t_ref, brel_ref, wroot_t_ref)
    o_ref[0] = res[:, :HD]
    o_ref[1] = res[:, HD:]


def _tc_layer(partials, h_split, wrel_t, brel, wroot_t):
    return pl.pallas_call(
        _tc_layer_body,
        grid=(N // BLK,),
        in_specs=[
            pl.BlockSpec((2, 2, BLK, HD), lambda i: (0, 0, i, 0)),
            pl.BlockSpec((2, BLK, HD), lambda i: (0, i, 0)),
            pl.BlockSpec((D, D), lambda i: (0, 0)),
            pl.BlockSpec((1, D), lambda i: (0, 0)),
            pl.BlockSpec((D, D), lambda i: (0, 0)),
        ],
        out_specs=pl.BlockSpec((2, BLK, HD), lambda i: (0, i, 0)),
        out_shape=jax.ShapeDtypeStruct((2, N, HD), jnp.float32),
    )(partials, h_split, wrel_t, brel.reshape(1, D), wroot_t)


def _tc_final_body(p_ref, h_ref, wrel_t_ref, brel_ref, wroot_t_ref,
                   batch_ref, wlin_ref, blin_ref, o_ref, acc_ref):
    i = pl.program_id(0)
    h3 = _assemble(p_ref, h_ref, wrel_t_ref, brel_ref, wroot_t_ref)
    b = batch_ref[0]
    onehot = (b.reshape(BLK, 1) ==
              lax.broadcasted_iota(jnp.int32, (BLK, G), 1)).astype(jnp.bfloat16)
    hi = h3.astype(jnp.bfloat16)
    lo = (h3 - hi.astype(jnp.float32)).astype(jnp.bfloat16)
    dn = (((0,), (0,)), ((), ()))
    part = (lax.dot_general(onehot, hi, dn, preferred_element_type=jnp.float32)
            + lax.dot_general(onehot, lo, dn, preferred_element_type=jnp.float32))

    @pl.when(i == 0)
    def _():
        acc_ref[...] = jnp.zeros_like(acc_ref)

    acc_ref[...] += part

    @pl.when(i == pl.num_programs(0) - 1)
    def _():
        pr = acc_ref[...].astype(jnp.bfloat16).astype(jnp.float32)
        wr = wlin_ref[...].astype(jnp.bfloat16).astype(jnp.float32)
        z = jnp.sum(pr * wr, axis=1, keepdims=True)
        o_ref[...] = jax.nn.sigmoid(z + blin_ref[0, 0])


def _tc_final(partials, h_split, wrel_t, brel, wroot_t, batch_r, wlin, blin):
    return pl.pallas_call(
        _tc_final_body,
        grid=(N // BLK,),
        in_specs=[
            pl.BlockSpec((2, 2, BLK, HD), lambda i: (0, 0, i, 0)),
            pl.BlockSpec((2, BLK, HD), lambda i: (0, i, 0)),
            pl.BlockSpec((D, D), lambda i: (0, 0)),
            pl.BlockSpec((1, D), lambda i: (0, 0)),
            pl.BlockSpec((D, D), lambda i: (0, 0)),
            pl.BlockSpec((1, 1, BLK), lambda i: (i, 0, 0)),
            pl.BlockSpec((1, D), lambda i: (0, 0)),
            pl.BlockSpec((1, 1), lambda i: (0, 0)),
        ],
        out_specs=pl.BlockSpec((G, 1), lambda i: (0, 0)),
        out_shape=jax.ShapeDtypeStruct((G, 1), jnp.float32),
        scratch_shapes=[pltpu.VMEM((G, D), jnp.float32)],
    )(partials, h_split, wrel_t, brel.reshape(1, D), wroot_t, batch_r,
      wlin.reshape(1, D), blin.reshape(1, 1))


def kernel(x, edge_index, batch, Wrel0, brel0, Wroot0, Wrel1, brel1, Wroot1,
           Wrel2, brel2, Wroot2, Wlin, blin):
    src = edge_index[0]
    dst = edge_index[1]
    pad = E_PAD - E
    src_r = jnp.concatenate([src, jnp.zeros((pad,), jnp.int32)]).reshape(
        NUM_TILES, CH_PER_TILE, CHUNK)
    dst_r = jnp.concatenate([dst, jnp.full((pad,), N, jnp.int32)]).reshape(
        NUM_TILES, CH_PER_TILE, CHUNK)
    batch_r = batch.reshape(N // BLK, 1, BLK)

    h = jnp.stack([x[:, :HD], x[:, HD:]])
    params = [(Wrel0, brel0, Wroot0), (Wrel1, brel1, Wroot1)]
    for (Wr, br, Wq) in params:
        partials = _sc_segsum(h, src_r, dst_r)
        h = _tc_layer(partials[:, :, :N, :], h, Wr.T, br, Wq.T)
    partials = _sc_segsum(h, src_r, dst_r)
    return _tc_final(partials[:, :, :N, :], h, Wrel2.T, brel2, Wroot2.T,
                     batch_r, Wlin, blin)


# speedup vs baseline: 9.9566x; 3.2941x over previous
import jax
import jax.numpy as jnp
from jax import lax
from jax.experimental import pallas as pl
from jax.experimental.pallas import tpu as pltpu
from jax.experimental.pallas import tpu_sc as plsc

N = 10000
E = 320000
D = 128
HD = D // 2
G = 128

NUM_TILES = 32
CHUNK = 128
CH_PER_TILE = 80
EDGES_PER_TILE = CHUNK * CH_PER_TILE
E_PAD = NUM_TILES * EDGES_PER_TILE
ROWS_PER_TILE = 640
N_ACC = 16 * ROWS_PER_TILE


NBUF = 4


def _sc_segsum_body(h_hbm, src_hbm, dst_hbm, out_hbm,
                    src_v, dst_v, rows_v, agg_sh, gsems, ssems):
    c = lax.axis_index("c")
    s = lax.axis_index("s")
    wid = c * 16 + s
    base = s * ROWS_PER_TILE

    pltpu.sync_copy(src_hbm.at[wid], src_v)
    pltpu.sync_copy(dst_hbm.at[wid], dst_v)

    def _gather(hf, j, b):
        return pltpu.make_async_copy(hf.at[src_v.at[j]], rows_v.at[b],
                                     gsems.at[b])

    def _scatter_start(j, b):
        pltpu.async_copy(rows_v.at[b], agg_sh.at[dst_v.at[j]],
                         ssems.at[b], add=True)

    def _scatter_wait(j, b):
        pltpu.make_async_copy(rows_v.at[b], agg_sh.at[dst_v.at[j]],
                              ssems.at[b]).wait()

    for f in range(2):
        hf = h_hbm.at[f]

        def _zero(i, _):
            rows_v[0, i // (HD // 16), pl.ds((i % (HD // 16)) * 16, 16)] = (
                jnp.zeros((16,), jnp.float32))
            return 0

        lax.fori_loop(0, CHUNK * (HD // 16), _zero, 0)
        for k in range(ROWS_PER_TILE // CHUNK):
            pltpu.sync_copy(rows_v.at[0],
                            agg_sh.at[pl.ds(base + k * CHUNK, CHUNK)])
        plsc.subcore_barrier()

        for b in range(NBUF):
            _gather(hf, b, b).start()

        def _body(i, _):
            for b in range(NBUF):
                j = NBUF * i + b
                _gather(hf, j, b).wait()
                _scatter_start(j, b)

                @pl.when(j + NBUF < CH_PER_TILE)
                def _():
                    _scatter_wait(j, b)
                    _gather(hf, j + NBUF, b).start()
            return 0

        lax.fori_loop(0, CH_PER_TILE // NBUF, _body, 0)
        for b in range(NBUF):
            _scatter_wait(CH_PER_TILE - NBUF + b, b)

        plsc.subcore_barrier()
        pltpu.sync_copy(agg_sh.at[pl.ds(base, ROWS_PER_TILE)],
                        out_hbm.at[c, f, pl.ds(base, ROWS_PER_TILE)])


def _sc_segsum(h_split, src_r, dst_r):
    kern = pl.kernel(
        _sc_segsum_body,
        out_type=jax.ShapeDtypeStruct((2, 2, N_ACC, HD), jnp.float32),
        mesh=plsc.VectorSubcoreMesh(core_axis_name="c", subcore_axis_name="s"),
        compiler_params=pltpu.CompilerParams(use_tc_tiling_on_sc=False),
        scratch_types=[
            pltpu.VMEM((CH_PER_TILE, CHUNK), jnp.int32),
            pltpu.VMEM((CH_PER_TILE, CHUNK), jnp.int32),
            pltpu.VMEM((NBUF, CHUNK, HD), jnp.float32),
            pltpu.VMEM_SHARED((N_ACC, HD), jnp.float32),
            pltpu.SemaphoreType.DMA((NBUF,)),
            pltpu.SemaphoreType.DMA((NBUF,)),
        ],
    )
    return kern(h_split, src_r, dst_r)


BLK = 1000


def _bdot(a, b):
    return jnp.dot(a.astype(jnp.bfloat16), b.astype(jnp.bfloat16),
                   preferred_element_type=jnp.float32)


def _assemble(p_ref, h_ref, wrel_t_ref, brel_ref, wroot_t_ref):
    agg = jnp.concatenate([p_ref[0, 0] + p_ref[1, 0],
                           p_ref[0, 1] + p_ref[1, 1]], axis=-1)
    h = jnp.concatenate([h_ref[0], h_ref[1]], axis=-1)
    acc = _bdot(agg, wrel_t_ref[...]) + brel_ref[...] + _bdot(h, wroot_t_ref[...])
    return jnp.tanh(acc)


def _tc_layer_body(p_ref, h_ref, wrel_t_ref, brel_ref, wroot_t_ref, o_ref):
    res = _assemble(p_ref, h_ref, wrel_t_ref, brel_ref, wroot_t_ref)
    o_ref[0] = res[:, :HD]
    o_ref[1] = res[:, HD:]


def _tc_layer(partials, h_split, wrel_t, brel, wroot_t):
    return pl.pallas_call(
        _tc_layer_body,
        grid=(N // BLK,),
        in_specs=[
            pl.BlockSpec((2, 2, BLK, HD), lambda i: (0, 0, i, 0)),
            pl.BlockSpec((2, BLK, HD), lambda i: (0, i, 0)),
            pl.BlockSpec((D, D), lambda i: (0, 0)),
            pl.BlockSpec((1, D), lambda i: (0, 0)),
            pl.BlockSpec((D, D), lambda i: (0, 0)),
        ],
        out_specs=pl.BlockSpec((2, BLK, HD), lambda i: (0, i, 0)),
        out_shape=jax.ShapeDtypeStruct((2, N, HD), jnp.float32),
    )(partials, h_split, wrel_t, brel.reshape(1, D), wroot_t)


def _tc_final_body(p_ref, h_ref, wrel_t_ref, brel_ref, wroot_t_ref,
                   batch_ref, wlin_ref, blin_ref, o_ref, acc_ref):
    i = pl.program_id(0)
    h3 = _assemble(p_ref, h_ref, wrel_t_ref, brel_ref, wroot_t_ref)
    b = batch_ref[0]
    onehot = (b.reshape(BLK, 1) ==
              lax.broadcasted_iota(jnp.int32, (BLK, G), 1)).astype(jnp.bfloat16)
    hi = h3.astype(jnp.bfloat16)
    lo = (h3 - hi.astype(jnp.float32)).astype(jnp.bfloat16)
    dn = (((0,), (0,)), ((), ()))
    part = (lax.dot_general(onehot, hi, dn, preferred_element_type=jnp.float32)
            + lax.dot_general(onehot, lo, dn, preferred_element_type=jnp.float32))

    @pl.when(i == 0)
    def _():
        acc_ref[...] = jnp.zeros_like(acc_ref)

    acc_ref[...] += part

    @pl.when(i == pl.num_programs(0) - 1)
    def _():
        pr = acc_ref[...].astype(jnp.bfloat16).astype(jnp.float32)
        wr = wlin_ref[...].astype(jnp.bfloat16).astype(jnp.float32)
        z = jnp.sum(pr * wr, axis=1, keepdims=True)
        o_ref[...] = jax.nn.sigmoid(z + blin_ref[0, 0])


def _tc_final(partials, h_split, wrel_t, brel, wroot_t, batch_r, wlin, blin):
    return pl.pallas_call(
        _tc_final_body,
        grid=(N // BLK,),
        in_specs=[
            pl.BlockSpec((2, 2, BLK, HD), lambda i: (0, 0, i, 0)),
            pl.BlockSpec((2, BLK, HD), lambda i: (0, i, 0)),
            pl.BlockSpec((D, D), lambda i: (0, 0)),
            pl.BlockSpec((1, D), lambda i: (0, 0)),
            pl.BlockSpec((D, D), lambda i: (0, 0)),
            pl.BlockSpec((1, 1, BLK), lambda i: (i, 0, 0)),
            pl.BlockSpec((1, D), lambda i: (0, 0)),
            pl.BlockSpec((1, 1), lambda i: (0, 0)),
        ],
        out_specs=pl.BlockSpec((G, 1), lambda i: (0, 0)),
        out_shape=jax.ShapeDtypeStruct((G, 1), jnp.float32),
        scratch_shapes=[pltpu.VMEM((G, D), jnp.float32)],
    )(partials, h_split, wrel_t, brel.reshape(1, D), wroot_t, batch_r,
      wlin.reshape(1, D), blin.reshape(1, 1))


def kernel(x, edge_index, batch, Wrel0, brel0, Wroot0, Wrel1, brel1, Wroot1,
           Wrel2, brel2, Wroot2, Wlin, blin):
    src = edge_index[0]
    dst = edge_index[1]
    pad = E_PAD - E
    pad_iota = jnp.arange(pad, dtype=jnp.int32)
    src_r = jnp.concatenate([src, pad_iota % N]).reshape(
        NUM_TILES, CH_PER_TILE, CHUNK)
    dst_r = jnp.concatenate([dst, N + pad_iota % (N_ACC - N)]).reshape(
        NUM_TILES, CH_PER_TILE, CHUNK)
    batch_r = batch.reshape(N // BLK, 1, BLK)

    h = jnp.stack([x[:, :HD], x[:, HD:]])
    params = [(Wrel0, brel0, Wroot0), (Wrel1, brel1, Wroot1)]
    for (Wr, br, Wq) in params:
        partials = _sc_segsum(h, src_r, dst_r)
        h = _tc_layer(partials[:, :, :N, :], h, Wr.T, br, Wq.T)
    partials = _sc_segsum(h, src_r, dst_r)
    return _tc_final(partials[:, :, :N, :], h, Wrel2.T, brel2, Wroot2.T,
                     batch_r, Wlin, blin)


# speedup vs baseline: 10.8907x; 1.0938x over previous
import jax
import jax.numpy as jnp
from jax import lax
from jax.experimental import pallas as pl
from jax.experimental.pallas import tpu as pltpu
from jax.experimental.pallas import tpu_sc as plsc

N = 10000
E = 320000
D = 128
HD = D // 2
G = 128

NUM_TILES = 32
CHUNK = 128
CH_PER_TILE = 80
EDGES_PER_TILE = CHUNK * CH_PER_TILE
E_PAD = NUM_TILES * EDGES_PER_TILE
ROWS_PER_TILE = 640
N_ACC = 16 * ROWS_PER_TILE


NBUF = 4


def _sc_segsum_body(h_hbm, src_hbm, dst_hbm, out_hbm,
                    src_v, dst_v, rows_v, agg_sh, gsems, ssems):
    c = lax.axis_index("c")
    s = lax.axis_index("s")
    wid = c * 16 + s
    base = s * ROWS_PER_TILE

    pltpu.sync_copy(src_hbm.at[wid], src_v)
    pltpu.sync_copy(dst_hbm.at[wid], dst_v)

    def _gather(hf, j, b):
        return pltpu.make_async_copy(hf.at[src_v.at[j]], rows_v.at[b],
                                     gsems.at[b])

    def _scatter_start(j, b):
        pltpu.async_copy(rows_v.at[b], agg_sh.at[dst_v.at[j]],
                         ssems.at[b], add=True)

    def _scatter_wait(j, b):
        pltpu.make_async_copy(rows_v.at[b], agg_sh.at[dst_v.at[j]],
                              ssems.at[b]).wait()

    for f in range(2):
        hf = h_hbm.at[f]

        def _zero(i, _):
            rows_v[0, i // (HD // 16), pl.ds((i % (HD // 16)) * 16, 16)] = (
                jnp.zeros((16,), jnp.float32))
            return 0

        lax.fori_loop(0, CHUNK * (HD // 16), _zero, 0)
        for k in range(ROWS_PER_TILE // CHUNK):
            pltpu.sync_copy(rows_v.at[0],
                            agg_sh.at[pl.ds(base + k * CHUNK, CHUNK)])
        plsc.subcore_barrier()

        for b in range(NBUF):
            _gather(hf, b, b).start()

        def _body(i, _):
            for b in range(NBUF):
                j = NBUF * i + b
                _gather(hf, j, b).wait()
                _scatter_start(j, b)

                @pl.when(j + NBUF < CH_PER_TILE)
                def _():
                    _scatter_wait(j, b)
                    _gather(hf, j + NBUF, b).start()
            return 0

        lax.fori_loop(0, CH_PER_TILE // NBUF, _body, 0)
        for b in range(NBUF):
            _scatter_wait(CH_PER_TILE - NBUF + b, b)

        plsc.subcore_barrier()
        pltpu.sync_copy(agg_sh.at[pl.ds(base, ROWS_PER_TILE)],
                        out_hbm.at[c, f, pl.ds(base, ROWS_PER_TILE)])


def _sc_segsum(h_split, src_r, dst_r):
    kern = pl.kernel(
        _sc_segsum_body,
        out_type=jax.ShapeDtypeStruct((2, 2, N_ACC, HD), jnp.float32),
        mesh=plsc.VectorSubcoreMesh(core_axis_name="c", subcore_axis_name="s"),
        compiler_params=pltpu.CompilerParams(use_tc_tiling_on_sc=False),
        scratch_types=[
            pltpu.VMEM((CH_PER_TILE, CHUNK), jnp.int32),
            pltpu.VMEM((CH_PER_TILE, CHUNK), jnp.int32),
            pltpu.VMEM((NBUF, CHUNK, HD), jnp.float32),
            pltpu.VMEM_SHARED((N_ACC, HD), jnp.float32),
            pltpu.SemaphoreType.DMA((NBUF,)),
            pltpu.SemaphoreType.DMA((NBUF,)),
        ],
    )
    return kern(h_split, src_r, dst_r)


BLK = 1000


def _bdot(a, b):
    return jnp.dot(a.astype(jnp.bfloat16), b.astype(jnp.bfloat16),
                   preferred_element_type=jnp.float32)


def _assemble(p_ref, h_ref, wrel_t_ref, brel_ref, wroot_t_ref):
    agg = jnp.concatenate([p_ref[0, 0] + p_ref[1, 0],
                           p_ref[0, 1] + p_ref[1, 1]], axis=-1)
    h = jnp.concatenate([h_ref[0], h_ref[1]], axis=-1)
    acc = _bdot(agg, wrel_t_ref[...]) + brel_ref[...] + _bdot(h, wroot_t_ref[...])
    return jnp.tanh(acc)


def _tc_layer_body(p_ref, h_ref, wrel_t_ref, brel_ref, wroot_t_ref, o_ref):
    res = _assemble(p_ref, h_ref, wrel_t_ref, brel_ref, wroot_t_ref)
    o_ref[0] = res[:, :HD]
    o_ref[1] = res[:, HD:]


def _tc_layer(partials, h_split, wrel_t, brel, wroot_t):
    return pl.pallas_call(
        _tc_layer_body,
        grid=(N // BLK,),
        in_specs=[
            pl.BlockSpec((2, 2, BLK, HD), lambda i: (0, 0, i, 0)),
            pl.BlockSpec((2, BLK, HD), lambda i: (0, i, 0)),
            pl.BlockSpec((D, D), lambda i: (0, 0)),
            pl.BlockSpec((1, D), lambda i: (0, 0)),
            pl.BlockSpec((D, D), lambda i: (0, 0)),
        ],
        out_specs=pl.BlockSpec((2, BLK, HD), lambda i: (0, i, 0)),
        out_shape=jax.ShapeDtypeStruct((2, N, HD), jnp.float32),
    )(partials, h_split, wrel_t, brel.reshape(1, D), wroot_t)


def _tc_final_body(p_ref, h_ref, wrel_t_ref, brel_ref, wroot_t_ref,
                   batch_ref, wlin_ref, blin_ref, o_ref, acc_ref):
    i = pl.program_id(0)
    h3 = _assemble(p_ref, h_ref, wrel_t_ref, brel_ref, wroot_t_ref)
    b = batch_ref[0]
    onehot = (b.reshape(BLK, 1) ==
              lax.broadcasted_iota(jnp.int32, (BLK, G), 1)).astype(jnp.bfloat16)
    hi = h3.astype(jnp.bfloat16)
    lo = (h3 - hi.astype(jnp.float32)).astype(jnp.bfloat16)
    dn = (((0,), (0,)), ((), ()))
    part = (lax.dot_general(onehot, hi, dn, preferred_element_type=jnp.float32)
            + lax.dot_general(onehot, lo, dn, preferred_element_type=jnp.float32))

    @pl.when(i == 0)
    def _():
        acc_ref[...] = jnp.zeros_like(acc_ref)

    acc_ref[...] += part

    @pl.when(i == pl.num_programs(0) - 1)
    def _():
        pr = acc_ref[...].astype(jnp.bfloat16).astype(jnp.float32)
        wr = wlin_ref[...].astype(jnp.bfloat16).astype(jnp.float32)
        z = jnp.sum(pr * wr, axis=1, keepdims=True)
        o_ref[...] = jax.nn.sigmoid(z + blin_ref[0, 0])


def _tc_final(partials, h_split, wrel_t, brel, wroot_t, batch_r, wlin, blin):
    return pl.pallas_call(
        _tc_final_body,
        grid=(N // BLK,),
        in_specs=[
            pl.BlockSpec((2, 2, BLK, HD), lambda i: (0, 0, i, 0)),
            pl.BlockSpec((2, BLK, HD), lambda i: (0, i, 0)),
            pl.BlockSpec((D, D), lambda i: (0, 0)),
            pl.BlockSpec((1, D), lambda i: (0, 0)),
            pl.BlockSpec((D, D), lambda i: (0, 0)),
            pl.BlockSpec((1, 1, BLK), lambda i: (i, 0, 0)),
            pl.BlockSpec((1, D), lambda i: (0, 0)),
            pl.BlockSpec((1, 1), lambda i: (0, 0)),
        ],
        out_specs=pl.BlockSpec((G, 1), lambda i: (0, 0)),
        out_shape=jax.ShapeDtypeStruct((G, 1), jnp.float32),
        scratch_shapes=[pltpu.VMEM((G, D), jnp.float32)],
    )(partials, h_split, wrel_t, brel.reshape(1, D), wroot_t, batch_r,
      wlin.reshape(1, D), blin.reshape(1, 1))


def kernel(x, edge_index, batch, Wrel0, brel0, Wroot0, Wrel1, brel1, Wroot1,
           Wrel2, brel2, Wroot2, Wlin, blin):
    src = edge_index[0]
    dst = edge_index[1]
    pad = E_PAD - E
    pad_iota = jnp.arange(pad, dtype=jnp.int32)
    src_r = jnp.concatenate([src, pad_iota % N]).reshape(
        NUM_TILES, CH_PER_TILE, CHUNK)
    dst_r = jnp.concatenate([dst, N + pad_iota % (N_ACC - N)]).reshape(
        NUM_TILES, CH_PER_TILE, CHUNK)
    batch_r = batch.reshape(N // BLK, 1, BLK)

    h = jnp.stack([x[:, :HD], x[:, HD:]])
    params = [(Wrel0, brel0, Wroot0), (Wrel1, brel1, Wroot1)]
    for (Wr, br, Wq) in params:
        partials = _sc_segsum(h, src_r, dst_r)
        h = _tc_layer(partials, h, Wr.T, br, Wq.T)
    partials = _sc_segsum(h, src_r, dst_r)
    return _tc_final(partials, h, Wrel2.T, brel2, Wroot2.T,
                     batch_r, Wlin, blin)
